# baseline (speedup 1.0000x reference)
import jax
import jax.numpy as jnp
import numpy as np
from jax import lax
from jax.experimental import pallas as pl
from jax.experimental.pallas import tpu as pltpu
from jax.experimental.pallas import tpu_sc as plsc

VOCAB = 100000
EMBED = 128
BATCH = 4096
HIST = 200

NWORKERS = 32
BPW = BATCH // NWORKERS
LOMASK = np.int32(0xFFFF)
HIMASK = np.int32(-65536)


def _body(ids_hbm, weight_hbm, out_hbm, ids_all, idx_v, rows_v, out_v,
          gsem0, gsem1, osem):
    wid = lax.axis_index("s") * 2 + lax.axis_index("c")
    base_b = wid * BPW
    weight_i32 = weight_hbm.bitcast(jnp.int32)
    gsems = (gsem0, gsem1)

    iota = lax.iota(jnp.int32, 16)
    c_tail = jnp.minimum(96 + iota, 99)
    tail_mask = iota < (HIST - 192)

    pltpu.sync_copy(ids_hbm.at[pl.ds(base_b, BPW)], ids_all)

    def prepare(bn, buf):
        for r in range(2):
            for c in range(6):
                v = ids_all[bn, r, pl.ds(c * 16, 16)]
                idx_v[buf, r, pl.ds(c * 16, 16)] = (
                    lax.shift_right_logical(v, 1))
            v = plsc.load_gather(
                ids_all, [jnp.full((16,), bn, jnp.int32),
                          jnp.full((16,), r, jnp.int32), c_tail])
            idx_v[buf, r, pl.ds(96, 16)] = lax.shift_right_logical(v, 1)
        for r in range(2):
            pltpu.async_copy(
                weight_i32.at[idx_v.at[buf, r, pl.ds(0, 100)]],
                rows_v.at[buf, pl.ds(r * 100, 100)], gsems[buf])

    def transpose(bn, buf):
        bsp = jnp.full((16,), bn, jnp.int32)
        for g in range(13):
            lvec = jnp.minimum(g * 16 + iota, HIST - 1)
            rrv = (lvec >= 100).astype(jnp.int32)
            ccv = lvec - 100 * rrv
            raw = plsc.load_gather(ids_all, [bsp, rrv, ccv])
            shv1 = lax.shift_left(raw & 1, 4)
            shv2 = 16 - shv1

            @plsc.parallel_loop(0, EMBED // 2, unroll=4)
            def per_d2(d2):
                ca = jnp.full((16,), 2 * d2, jnp.int32)
                va = plsc.load_gather(rows_v.at[buf], [lvec, ca])
                vb = plsc.load_gather(rows_v.at[buf], [lvec, ca + 1])
                lo = lax.shift_right_logical(va, shv1) & LOMASK
                hi = lax.shift_left(vb, shv2) & HIMASK
                word = lo | hi
                if g < 12:
                    out_v[buf, d2, pl.ds(g * 16, 16)] = word
                else:
                    plsc.store_scatter(
                        out_v.at[buf],
                        [jnp.full((16,), d2, jnp.int32), lvec],
                        word, mask=tail_mask)

    def wait_gather(buf):
        pltpu.make_async_copy(
            weight_i32.at[pl.ds(0, HIST)], rows_v.at[buf],
            gsems[buf]).wait()

    def drain_out(buf):
        pltpu.make_async_copy(
            out_hbm.at[base_b], out_v.at[buf].bitcast(jnp.float16),
            osem).wait()

    prepare(0, 0)

    def step(k, _):
        for buf in range(2):
            bi = 2 * k + buf
            prepare(jnp.minimum(bi + 1, BPW - 1), 1 - buf)
            wait_gather(buf)

            @pl.when(k >= 1)
            def _drain():
                drain_out(buf)

            transpose(bi, buf)
            pltpu.async_copy(
                out_v.at[buf].bitcast(jnp.float16),
                out_hbm.at[base_b + bi], osem)
        return _

    lax.fori_loop(0, BPW // 2, step, None)

    wait_gather(0)
    drain_out(0)
    drain_out(1)


@jax.jit
def kernel(ids, weight):
    ids3 = ids.astype(jnp.int32).reshape(BATCH, 2, HIST // 2)

    mesh = plsc.VectorSubcoreMesh(core_axis_name="c", subcore_axis_name="s")
    return pl.kernel(
        _body,
        out_type=jax.ShapeDtypeStruct((BATCH, EMBED, HIST), jnp.float16),
        mesh=mesh,
        scratch_types=[
            pltpu.VMEM((BPW, 2, HIST // 2), jnp.int32),
            pltpu.VMEM((2, 2, 112), jnp.int32),
            pltpu.VMEM((2, HIST, EMBED), jnp.int32),
            pltpu.VMEM((2, EMBED // 2, HIST), jnp.int32),
            pltpu.SemaphoreType.DMA,
            pltpu.SemaphoreType.DMA,
            pltpu.SemaphoreType.DMA,
        ],
        compiler_params=pltpu.CompilerParams(needs_layout_passes=False),
    )(ids3, weight)


# speedup vs baseline: 1.5218x; 1.5218x over previous
import jax
import jax.numpy as jnp
import numpy as np
from jax import lax
from jax.experimental import pallas as pl
from jax.experimental.pallas import tpu as pltpu
from jax.experimental.pallas import tpu_sc as plsc

VOCAB = 100000
EMBED = 128
BATCH = 4096
HIST = 200

NWORKERS = 32
BPW = BATCH // NWORKERS
NCHUNK = (HIST + 15) // 16
NTAIL = HIST - (NCHUNK - 1) * 16
VBLK = 1000


def _repack_body(w_ref, y_ref):
    xu = w_ref[...].astype(jnp.int32) & np.int32(0xFFFF)
    xn = jnp.concatenate([xu[:, 1:], xu[:, :1]], axis=1)
    y_ref[...] = xu | lax.shift_left(xn, 16)


def _sc_body(ids_hbm, y_hbm, out_hbm, ids_all, rows_v, out_v,
             gsem0, gsem1, osem):
    wid = lax.axis_index("s") * 2 + lax.axis_index("c")
    base_b = wid * BPW
    gsems = (gsem0, gsem1)

    iota = lax.iota(jnp.int32, 16)
    l_idx = [jnp.minimum(lc * 16 + iota, HIST - 1) for lc in range(NCHUNK)]
    tail_mask = iota < NTAIL

    pltpu.sync_copy(ids_hbm.at[pl.ds(base_b, BPW)], ids_all)

    def prepare(bn, buf):
        for r in range(2):
            pltpu.async_copy(
                y_hbm.at[ids_all.at[bn, r]],
                rows_v.at[buf, pl.ds(r * 100, 100)], gsems[buf])

    def transpose(buf):
        @plsc.parallel_loop(0, EMBED // 2, unroll=8)
        def per_dpair(d2):
            col = jnp.full((16,), 2 * d2, jnp.int32)
            for lc in range(NCHUNK):
                word = plsc.load_gather(rows_v.at[buf], [l_idx[lc], col])
                if lc < NCHUNK - 1:
                    out_v[buf, d2, pl.ds(lc * 16, 16)] = word
                else:
                    plsc.store_scatter(
                        out_v.at[buf],
                        [jnp.full((16,), d2, jnp.int32), l_idx[lc]],
                        word, mask=tail_mask)

    def wait_gather(buf):
        pltpu.make_async_copy(
            y_hbm.at[pl.ds(0, HIST)], rows_v.at[buf], gsems[buf]).wait()

    def drain_out(buf):
        pltpu.make_async_copy(
            out_hbm.at[base_b], out_v.at[buf].bitcast(jnp.float16),
            osem).wait()

    prepare(0, 0)

    def step(k, _):
        for buf in range(2):
            bi = 2 * k + buf
            prepare(jnp.minimum(bi + 1, BPW - 1), 1 - buf)
            wait_gather(buf)

            @pl.when(k >= 1)
            def _drain():
                drain_out(buf)

            transpose(buf)
            pltpu.async_copy(
                out_v.at[buf].bitcast(jnp.float16),
                out_hbm.at[base_b + bi], osem)
        return _

    lax.fori_loop(0, BPW // 2, step, None)

    wait_gather(0)
    drain_out(0)
    drain_out(1)


@jax.jit
def kernel(ids, weight):
    ids3 = ids.astype(jnp.int32).reshape(BATCH, 2, HIST // 2)

    wu = lax.bitcast_convert_type(weight, jnp.uint16)
    y = pl.pallas_call(
        _repack_body,
        grid=(VOCAB // VBLK,),
        in_specs=[pl.BlockSpec((VBLK, EMBED), lambda i: (i, 0))],
        out_specs=pl.BlockSpec((VBLK, EMBED), lambda i: (i, 0)),
        out_shape=jax.ShapeDtypeStruct((VOCAB, EMBED), jnp.int32),
    )(wu)

    mesh = plsc.VectorSubcoreMesh(core_axis_name="c", subcore_axis_name="s")
    return pl.kernel(
        _sc_body,
        out_type=jax.ShapeDtypeStruct((BATCH, EMBED, HIST), jnp.float16),
        mesh=mesh,
        scratch_types=[
            pltpu.VMEM((BPW, 2, HIST // 2), jnp.int32),
            pltpu.VMEM((2, HIST, EMBED), jnp.int32),
            pltpu.VMEM((2, EMBED // 2, HIST), jnp.int32),
            pltpu.SemaphoreType.DMA,
            pltpu.SemaphoreType.DMA,
            pltpu.SemaphoreType.DMA,
        ],
        compiler_params=pltpu.CompilerParams(needs_layout_passes=False),
    )(ids3, y)


# speedup vs baseline: 2.3432x; 1.5397x over previous
import jax
import jax.numpy as jnp
import numpy as np
from jax import lax
from jax.experimental import pallas as pl
from jax.experimental.pallas import tpu as pltpu
from jax.experimental.pallas import tpu_sc as plsc

VOCAB = 100000
EMBED = 128
BATCH = 4096
HIST = 200

NWORKERS = 32
BPW = BATCH // NWORKERS
NCHUNK = (HIST + 15) // 16
NTAIL = HIST - (NCHUNK - 1) * 16
VBLK = 1000


def _repack_body(w_ref, y_ref):
    xu = w_ref[...].astype(jnp.int32) & np.int32(0xFFFF)
    xn = jnp.concatenate([xu[:, 1:], xu[:, :1]], axis=1)
    y_ref[...] = xu | lax.shift_left(xn, 16)


def _sc_body(ids_hbm, y_hbm, out_hbm, ids_all, rows_v, out_v,
             gsem0, gsem1, osem):
    wid = lax.axis_index("s") * 2 + lax.axis_index("c")
    base_b = wid * BPW
    gsems = (gsem0, gsem1)

    iota = lax.iota(jnp.int32, 16)
    l_idx = [jnp.minimum(lc * 16 + iota, HIST - 1) for lc in range(NCHUNK)]
    tail_mask = iota < NTAIL

    pltpu.sync_copy(ids_hbm.at[pl.ds(base_b, BPW)], ids_all)

    def prepare(bn, buf):
        for r in range(2):
            pltpu.async_copy(
                y_hbm.at[ids_all.at[bn, r]],
                rows_v.at[buf, pl.ds(r * 100, 100)], gsems[buf])

    def transpose(buf):
        for c in range(4):
            @plsc.parallel_loop(0, NCHUNK, unroll=1)
            def per_group(g):
                lvec_raw = g * 16 + iota
                mask = lvec_raw < HIST
                lvec = jnp.minimum(lvec_raw, HIST - 1)
                for o in range(16):
                    perm = (iota + o) & 15
                    colv = 32 * c + 2 * perm
                    word = plsc.load_gather(rows_v.at[buf], [lvec, colv],
                                            mask=mask)
                    plsc.store_scatter(out_v.at[buf],
                                       [16 * c + perm, lvec],
                                       word, mask=mask)

    def wait_gather(buf):
        pltpu.make_async_copy(
            y_hbm.at[pl.ds(0, HIST)], rows_v.at[buf], gsems[buf]).wait()

    def drain_out(buf):
        pltpu.make_async_copy(
            out_hbm.at[base_b], out_v.at[buf].bitcast(jnp.float16),
            osem).wait()

    prepare(0, 0)

    def step(k, _):
        for buf in range(2):
            bi = 2 * k + buf
            prepare(jnp.minimum(bi + 1, BPW - 1), 1 - buf)
            wait_gather(buf)

            @pl.when(k >= 1)
            def _drain():
                drain_out(buf)

            transpose(buf)
            pltpu.async_copy(
                out_v.at[buf].bitcast(jnp.float16),
                out_hbm.at[base_b + bi], osem)
        return _

    lax.fori_loop(0, BPW // 2, step, None)

    wait_gather(0)
    drain_out(0)
    drain_out(1)


@jax.jit
def kernel(ids, weight):
    ids3 = ids.astype(jnp.int32).reshape(BATCH, 2, HIST // 2)

    wu = lax.bitcast_convert_type(weight, jnp.uint16)
    y = pl.pallas_call(
        _repack_body,
        grid=(VOCAB // VBLK,),
        in_specs=[pl.BlockSpec((VBLK, EMBED), lambda i: (i, 0))],
        out_specs=pl.BlockSpec((VBLK, EMBED), lambda i: (i, 0)),
        out_shape=jax.ShapeDtypeStruct((VOCAB, EMBED), jnp.int32),
    )(wu)

    mesh = plsc.VectorSubcoreMesh(core_axis_name="c", subcore_axis_name="s")
    out_sc = pl.kernel(
        _sc_body,
        out_type=jax.ShapeDtypeStruct((BATCH, EMBED, HIST), jnp.float16),
        mesh=mesh,
        scratch_types=[
            pltpu.VMEM((BPW, 2, HIST // 2), jnp.int32),
            pltpu.VMEM((2, HIST, EMBED), jnp.int32),
            pltpu.VMEM((2, EMBED // 2, HIST), jnp.int32),
            pltpu.SemaphoreType.DMA,
            pltpu.SemaphoreType.DMA,
            pltpu.SemaphoreType.DMA,
        ],
        compiler_params=pltpu.CompilerParams(needs_layout_passes=False),
    )(ids3, y)

    return out_sc
